# Initial kernel scaffold; baseline (speedup 1.0000x reference)
import jax
import jax.numpy as jnp
from jax.experimental import pallas as pl


def kernel(input, target, weight):
    raise NotImplementedError("write your pallas kernel here")


# speedup vs baseline: 1.0012x; 1.0012x over previous
import jax
import jax.numpy as jnp
from jax.experimental import pallas as pl
from jax.experimental.pallas import tpu as pltpu

_N = 262144
_EPS = 1e-7
_R = 512


def _loss_body(lh_ref, ev_ref, out_ref):
    lh = lh_ref[...]
    ev = ev_ref[...]
    gamma = jnp.max(lh)
    ex = jnp.exp(lh - gamma)

    k = jax.lax.broadcasted_iota(jnp.int32, (_R, _R), 0)
    c = jax.lax.broadcasted_iota(jnp.int32, (_R, _R), 1)
    incl = (k <= c).astype(jnp.float32)
    strict = (c < k).astype(jnp.float32)

    row_cum = jax.lax.dot_general(
        ex, incl, (((1,), (0,)), ((), ())),
        precision=jax.lax.Precision.HIGHEST,
        preferred_element_type=jnp.float32)
    tot = row_cum[:, _R - 1:_R]
    carry = jax.lax.dot_general(
        strict, tot, (((1,), (0,)), ((), ())),
        precision=jax.lax.Precision.HIGHEST,
        preferred_element_type=jnp.float32)
    cum = row_cum + carry
    log_cum = jnp.log(cum + _EPS) + gamma
    se = jnp.sum(ev)
    num = jnp.sum((lh - log_cum) * ev)
    out_ref[...] = jnp.reshape(-num / se, (1, 1))


def kernel(input, target, weight):
    idx = jnp.argsort(-target)
    lh = input[idx].reshape(_R, _R)
    ev = weight[idx].astype(jnp.float32).reshape(_R, _R)
    out = pl.pallas_call(
        _loss_body,
        out_shape=jax.ShapeDtypeStruct((1, 1), jnp.float32),
    )(lh, ev)
    return out[0, 0]
